# baseline (speedup 1.0000x reference)
import functools

import jax
import jax.numpy as jnp
from jax import lax
from jax.experimental import pallas as pl
from jax.experimental.pallas import tpu as pltpu
from jax.experimental.pallas import tpu_sc as plsc

BATCH = 4096
SEQ = 50
EMBED = 64
HIDDEN = 128
LABELS = 32
VOCAB = 100000

NC = 2
NS = 16
NW = NC * NS
DIMS_PER_W = EMBED // NW
BB = 128
IDS_PER_BLK = BB * SEQ
NBLK = BATCH // BB
GRP = BB // 16


def _sc_pool_body(ids_hbm, tableT_hbm, out_hbm, trow_v, idsb0, idsb1,
                  acc_v, *sems):
    c = lax.axis_index("c")
    s = lax.axis_index("s")
    wid = s * NC + c
    lane_off = lax.iota(jnp.int32, 16) * SEQ
    zero16 = jnp.zeros((16,), jnp.int32)

    idsb = (idsb0, idsb1)

    def ids_start(blk, b):
        pltpu.async_copy(ids_hbm.at[pl.ds(blk * IDS_PER_BLK, IDS_PER_BLK)],
                         idsb[b], sems[b])

    def ids_wait(b):
        pltpu.make_async_copy(ids_hbm.at[pl.ds(0, IDS_PER_BLK)],
                              idsb[b], sems[b]).wait()

    for dd in range(DIMS_PER_W):
        d = wid * DIMS_PER_W + dd
        pltpu.sync_copy(tableT_hbm.at[pl.ds(d, 1)], trow_v)
        ids_start(0, 0)
        ids_start(1, 1)

        def pair(i, carry):
            for b in range(2):
                blk = i * 2 + b
                ids_wait(b)
                for g in range(GRP):
                    base = lane_off + (g * (16 * SEQ))
                    a0 = jnp.zeros((16,), jnp.float32)
                    a1 = jnp.zeros((16,), jnp.float32)
                    for t in range(0, SEQ, 2):
                        iv0 = plsc.load_gather(idsb[b], [base + t])
                        a0 = a0 + plsc.load_gather(trow_v, [zero16, iv0])
                        iv1 = plsc.load_gather(idsb[b], [base + (t + 1)])
                        a1 = a1 + plsc.load_gather(trow_v, [zero16, iv1])
                    acc_v[pl.ds(blk * BB + g * 16, 16)] = a0 + a1
                nxt = blk + 2

                @pl.when(nxt < NBLK)
                def _():
                    ids_start(nxt, b)

            return carry

        lax.fori_loop(0, NBLK // 2, pair, 0)
        pltpu.sync_copy(acc_v, out_hbm.at[pl.ds(d * BATCH, BATCH)])


_sc_pool = functools.partial(
    pl.kernel,
    out_type=jax.ShapeDtypeStruct((EMBED * BATCH,), jnp.float32),
    mesh=plsc.VectorSubcoreMesh(core_axis_name="c", subcore_axis_name="s"),
    scratch_types=[
        pltpu.VMEM((1, VOCAB), jnp.float32),
        pltpu.VMEM((IDS_PER_BLK,), jnp.int32),
        pltpu.VMEM((IDS_PER_BLK,), jnp.int32),
        pltpu.VMEM((BATCH,), jnp.float32),
        pltpu.SemaphoreType.DMA,
        pltpu.SemaphoreType.DMA,
    ],
    compiler_params=pltpu.CompilerParams(needs_layout_passes=False),
)(_sc_pool_body)


def _tc_head_body(idsT_ref, psumT_ref, gamma_ref, beta_ref, w1_ref, b1_ref,
                  w2_ref, b2_ref, outT_ref):
    idsT = idsT_ref[...]
    cnt = jnp.sum((idsT != 0).astype(jnp.float32), axis=0, keepdims=True)
    pooled = psumT_ref[...] / jnp.maximum(cnt, 1.0)
    mean = jnp.mean(pooled, axis=0, keepdims=True)
    centered = pooled - mean
    var = jnp.mean(centered * centered, axis=0, keepdims=True)
    normed = (centered * lax.rsqrt(var + 1e-5) * gamma_ref[...]
              + beta_ref[...])
    h = lax.dot_general(w1_ref[...], normed, (((0,), (0,)), ((), ())),
                        preferred_element_type=jnp.float32)
    h = jnp.maximum(h + b1_ref[...], 0.0)
    out = lax.dot_general(w2_ref[...], h, (((0,), (0,)), ((), ())),
                          preferred_element_type=jnp.float32)
    outT_ref[...] = out + b2_ref[...]


def _tc_head(idsT, psumT, gamma, beta, W1, b1, W2, b2):
    blk = 512
    grid = BATCH // blk
    return pl.pallas_call(
        _tc_head_body,
        grid=(grid,),
        in_specs=[
            pl.BlockSpec((SEQ, blk), lambda i: (0, i)),
            pl.BlockSpec((EMBED, blk), lambda i: (0, i)),
            pl.BlockSpec((EMBED, 1), lambda i: (0, 0)),
            pl.BlockSpec((EMBED, 1), lambda i: (0, 0)),
            pl.BlockSpec((EMBED, HIDDEN), lambda i: (0, 0)),
            pl.BlockSpec((HIDDEN, 1), lambda i: (0, 0)),
            pl.BlockSpec((HIDDEN, LABELS), lambda i: (0, 0)),
            pl.BlockSpec((LABELS, 1), lambda i: (0, 0)),
        ],
        out_specs=pl.BlockSpec((LABELS, blk), lambda i: (0, i)),
        out_shape=jax.ShapeDtypeStruct((LABELS, BATCH), jnp.float32),
    )(idsT, psumT, gamma, beta, W1, b1, W2, b2)


def kernel(token_ids, table, gamma, beta, W1, b1, W2, b2):
    ids32 = token_ids.astype(jnp.int32)
    psumT = _sc_pool(ids32.reshape(-1), table.T).reshape(EMBED, BATCH)
    outT = _tc_head(ids32.T, psumT,
                    gamma.reshape(EMBED, 1), beta.reshape(EMBED, 1),
                    W1, b1.reshape(HIDDEN, 1), W2, b2.reshape(LABELS, 1))
    return outT.T


# speedup vs baseline: 1.3870x; 1.3870x over previous
import functools

import jax
import jax.numpy as jnp
from jax import lax
from jax.experimental import pallas as pl
from jax.experimental.pallas import tpu as pltpu
from jax.experimental.pallas import tpu_sc as plsc

BATCH = 4096
SEQ = 50
EMBED = 64
HIDDEN = 128
LABELS = 32

NC = 2
NS = 16
NW = NC * NS
ROWS_PER_W = BATCH // NW
CHUNKS = ROWS_PER_W
NBUF = 8


def _sc_pool_body(ids_hbm, table_hbm, out_hbm, idx_v, rows_v, acc_v, *sems):
    c = lax.axis_index("c")
    s = lax.axis_index("s")
    wid = s * NC + c
    pltpu.sync_copy(ids_hbm.at[pl.ds(wid * ROWS_PER_W, ROWS_PER_W)], idx_v)

    def start(chunk, b):
        pltpu.async_copy(table_hbm.at[idx_v.at[chunk]], rows_v.at[b], sems[b])

    def wait(b):
        pltpu.make_async_copy(
            table_hbm.at[idx_v.at[0]], rows_v.at[b], sems[b]
        ).wait()

    for b in range(NBUF):
        start(b, b)

    def group(g, carry):
        for b in range(NBUF):
            cur = g * NBUF + b
            wait(b)
            for cc in range(EMBED // 16):
                sl = pl.ds(cc * 16, 16)
                a = [rows_v[b, t, sl] for t in range(4)]
                for t in range(4, SEQ - 2, 4):
                    for j in range(4):
                        a[j] = a[j] + rows_v[b, t + j, sl]
                a[0] = a[0] + rows_v[b, SEQ - 2, sl]
                a[1] = a[1] + rows_v[b, SEQ - 1, sl]
                acc_v[cur, sl] = (a[0] + a[1]) + (a[2] + a[3])
            nxt = cur + NBUF

            @pl.when(nxt < CHUNKS)
            def _():
                start(nxt, b)

        return carry

    lax.fori_loop(0, CHUNKS // NBUF, group, 0)
    pltpu.sync_copy(acc_v, out_hbm.at[pl.ds(wid * ROWS_PER_W, ROWS_PER_W)])


_sc_pool = functools.partial(
    pl.kernel,
    out_type=jax.ShapeDtypeStruct((BATCH, EMBED), jnp.float32),
    mesh=plsc.VectorSubcoreMesh(core_axis_name="c", subcore_axis_name="s"),
    scratch_types=[
        pltpu.VMEM((ROWS_PER_W, SEQ), jnp.int32),
        pltpu.VMEM((NBUF, SEQ, EMBED), jnp.float32),
        pltpu.VMEM((ROWS_PER_W, EMBED), jnp.float32),
    ] + [pltpu.SemaphoreType.DMA] * NBUF,
    compiler_params=pltpu.CompilerParams(use_tc_tiling_on_sc=False),
)(_sc_pool_body)


def _tc_head_body(ids_ref, psum_ref, gamma_ref, beta_ref, w1_ref, b1_ref,
                  w2_ref, b2_ref, out_ref):
    ids = ids_ref[...]
    cnt = jnp.sum((ids != 0).astype(jnp.float32), axis=1, keepdims=True)
    pooled = psum_ref[...] / jnp.maximum(cnt, 1.0)
    mean = jnp.mean(pooled, axis=1, keepdims=True)
    centered = pooled - mean
    var = jnp.mean(centered * centered, axis=1, keepdims=True)
    normed = centered * lax.rsqrt(var + 1e-5) * gamma_ref[...] + beta_ref[...]
    h = jnp.dot(normed, w1_ref[...], preferred_element_type=jnp.float32)
    h = jnp.maximum(h + b1_ref[...], 0.0)
    out = jnp.dot(h, w2_ref[...], preferred_element_type=jnp.float32)
    out_ref[...] = out + b2_ref[...]


def _tc_head(token_ids, psum, gamma, beta, W1, b1, W2, b2):
    blk = 512
    grid = BATCH // blk
    return pl.pallas_call(
        _tc_head_body,
        grid=(grid,),
        in_specs=[
            pl.BlockSpec((blk, SEQ), lambda i: (i, 0)),
            pl.BlockSpec((blk, EMBED), lambda i: (i, 0)),
            pl.BlockSpec((1, EMBED), lambda i: (0, 0)),
            pl.BlockSpec((1, EMBED), lambda i: (0, 0)),
            pl.BlockSpec((EMBED, HIDDEN), lambda i: (0, 0)),
            pl.BlockSpec((1, HIDDEN), lambda i: (0, 0)),
            pl.BlockSpec((HIDDEN, LABELS), lambda i: (0, 0)),
            pl.BlockSpec((1, LABELS), lambda i: (0, 0)),
        ],
        out_specs=pl.BlockSpec((blk, LABELS), lambda i: (i, 0)),
        out_shape=jax.ShapeDtypeStruct((BATCH, LABELS), jnp.float32),
    )(token_ids, psum, gamma, beta, W1, b1, W2, b2)


def kernel(token_ids, table, gamma, beta, W1, b1, W2, b2):
    psum = _sc_pool(token_ids.astype(jnp.int32), table)
    return _tc_head(token_ids.astype(jnp.int32), psum,
                    gamma.reshape(1, EMBED), beta.reshape(1, EMBED),
                    W1, b1.reshape(1, HIDDEN), W2, b2.reshape(1, LABELS))
